# baseline (speedup 1.0000x reference)
import jax
import jax.numpy as jnp
from jax.experimental import pallas as pl
from jax.experimental.pallas import tpu as pltpu

_BM = 200
_NBUF = 4
_LAG = 3


def _gcn_kernel(v_ref, w_ref, adj_hbm, out_ref, vw_ref, buf, sems):
    i = pl.program_id(0)
    nblocks = pl.num_programs(0) - _LAG

    @pl.when(i == 0)
    def _():
        vw_ref[...] = jax.lax.dot_general(
            v_ref[...].astype(jnp.bfloat16), w_ref[...].astype(jnp.bfloat16),
            dimension_numbers=(((1,), (1,)), ((), ())),
            preferred_element_type=jnp.float32,
        ).astype(jnp.bfloat16)

    @pl.when(i < nblocks)
    def _():
        slot = jax.lax.rem(i, _NBUF)
        pltpu.make_async_copy(
            adj_hbm.at[pl.ds(i * _BM, _BM), :],
            buf.at[slot],
            sems.at[slot],
        ).start()

    @pl.when(i >= _LAG)
    def _():
        j = i - _LAG
        slot = jax.lax.rem(j, _NBUF)
        pltpu.make_async_copy(
            adj_hbm.at[pl.ds(j * _BM, _BM), :],
            buf.at[slot],
            sems.at[slot],
        ).wait()
        out_ref[...] = jnp.maximum(
            jnp.dot(buf[slot].astype(jnp.bfloat16), vw_ref[...],
                    preferred_element_type=jnp.float32),
            0.0,
        )


def kernel(v, adj, W):
    N, d_in = v.shape
    d_out = W.shape[0]
    nblocks = N // _BM

    out = pl.pallas_call(
        _gcn_kernel,
        grid=(nblocks + _LAG,),
        in_specs=[
            pl.BlockSpec((N, d_in), lambda i: (0, 0)),
            pl.BlockSpec((d_out, d_in), lambda i: (0, 0)),
            pl.BlockSpec(memory_space=pl.ANY),
        ],
        out_specs=pl.BlockSpec(
            (_BM, d_out), lambda i: (jnp.maximum(i - _LAG, 0), 0)
        ),
        out_shape=jax.ShapeDtypeStruct((N, d_out), jnp.float32),
        scratch_shapes=[
            pltpu.VMEM((N, d_out), jnp.bfloat16),
            pltpu.VMEM((_NBUF, _BM, N), jnp.float32),
            pltpu.SemaphoreType.DMA((_NBUF,)),
        ],
        compiler_params=pltpu.CompilerParams(
            dimension_semantics=("arbitrary",),
        ),
    )(v, W, adj)

    return (out, adj)


# speedup vs baseline: 1.0076x; 1.0076x over previous
import jax
import jax.numpy as jnp
from jax.experimental import pallas as pl
from jax.experimental.pallas import tpu as pltpu


def _gcn_kernel(v_ref, w_ref, adj_ref, out_ref, vw_ref):
    @pl.when(pl.program_id(0) == 0)
    def _():
        vw_ref[...] = jax.lax.dot_general(
            v_ref[...].astype(jnp.bfloat16), w_ref[...].astype(jnp.bfloat16),
            dimension_numbers=(((1,), (1,)), ((), ())),
            preferred_element_type=jnp.float32,
        ).astype(jnp.bfloat16)

    out_ref[...] = jnp.maximum(
        jnp.dot(adj_ref[...].astype(jnp.bfloat16), vw_ref[...],
                preferred_element_type=jnp.float32),
        0.0,
    )


def kernel(v, adj, W):
    N, d_in = v.shape
    d_out = W.shape[0]

    BM = 400
    out = pl.pallas_call(
        _gcn_kernel,
        grid=(N // BM,),
        in_specs=[
            pl.BlockSpec((N, d_in), lambda i: (0, 0)),
            pl.BlockSpec((d_out, d_in), lambda i: (0, 0)),
            pl.BlockSpec((BM, N), lambda i: (i, 0)),
        ],
        out_specs=pl.BlockSpec((BM, d_out), lambda i: (i, 0)),
        out_shape=jax.ShapeDtypeStruct((N, d_out), jnp.float32),
        scratch_shapes=[pltpu.VMEM((N, d_out), jnp.bfloat16)],
        compiler_params=pltpu.CompilerParams(
            dimension_semantics=("arbitrary",),
        ),
    )(v, W, adj)

    return (out, adj)
